# Initial kernel scaffold; baseline (speedup 1.0000x reference)
import jax
import jax.numpy as jnp
from jax.experimental import pallas as pl


def kernel(edge_src, edge_dst, edge_type, dynamic_emb, emb_rel, weight_neighbor, loop_weight, evolve_loop_weight, time_gate_weight, time_gate_bias):
    raise NotImplementedError("write your pallas kernel here")


# speedup vs baseline: 2.5293x; 2.5293x over previous
import jax
import jax.numpy as jnp
from jax import lax
from jax.experimental import pallas as pl
from jax.experimental.pallas import tpu as pltpu
from jax.experimental.pallas import tpu_sc as plsc

_N = 10000
_H = 128
_T = 3
_E = 320000
_NPAD = 10112
_NC, _NS = 2, 16
_NW = _NC * _NS
_STRIPE = _NPAD // _NS
_BATCH = 128
_BPW = 80
_BPH = _BPW // 2
_NB = _NW * _BPW
_EPAD = _NB * _BATCH
_NBUF = 2
_GROUPS = _BPH // _NBUF
_SLOPE = (1.0 / 8.0 + 1.0 / 3.0) / 2.0
_RB = 1000
_GRID = _N // _RB


def _gather_body(table, src_idx, dst_idx, zeros_g, out_g,
                 acc_g, src_v, dst_v, b0_, b1_, s0_, s1_):
    bufs = (b0_, b1_)
    sems = (s0_, s1_)
    c = lax.axis_index("c")
    s = lax.axis_index("s")
    w = c * _NS + s
    row0 = s * _STRIPE
    pltpu.sync_copy(zeros_g.at[pl.ds(row0, _STRIPE)],
                    acc_g.at[pl.ds(row0, _STRIPE)])
    plsc.subcore_barrier()
    bb = w * _BPW
    for hf in range(2):
        pltpu.sync_copy(src_idx.at[pl.ds(bb + hf * _BPH, _BPH)], src_v)
        pltpu.sync_copy(dst_idx.at[pl.ds(bb + hf * _BPH, _BPH)], dst_v)
        for b in range(_NBUF):
            pltpu.async_copy(table.at[src_v.at[b]], bufs[b], sems[b])

        def group(k, carry):
            j0 = k * _NBUF
            for b in range(_NBUF):
                pltpu.make_async_copy(table.at[src_v.at[0]], bufs[b],
                                      sems[b]).wait()
                pltpu.sync_copy(bufs[b], acc_g.at[dst_v.at[j0 + b]], add=True)

                @pl.when(k < _GROUPS - 1)
                def _():
                    pltpu.async_copy(table.at[src_v.at[j0 + b + _NBUF]],
                                     bufs[b], sems[b])
            return carry

        lax.fori_loop(0, _GROUPS, group, 0)
    plsc.subcore_barrier()
    pltpu.sync_copy(acc_g.at[pl.ds(row0, _STRIPE)],
                    out_g.at[c, pl.ds(row0, _STRIPE)])


def _deg_body(dst_idx, zeros_d, ones_hbm, out_d,
              acc_d, dst_v, ones_v):
    c = lax.axis_index("c")
    s = lax.axis_index("s")
    w = c * _NS + s
    row0 = s * _STRIPE
    pltpu.sync_copy(zeros_d.at[pl.ds(row0, _STRIPE)],
                    acc_d.at[pl.ds(row0, _STRIPE)])
    pltpu.sync_copy(ones_hbm, ones_v)
    pltpu.sync_copy(dst_idx.at[pl.ds(w * _BPW, _BPW)], dst_v)
    plsc.subcore_barrier()

    def step(j, carry):
        pltpu.sync_copy(ones_v, acc_d.at[dst_v.at[j]], add=True)
        return carry

    lax.fori_loop(0, _BPW, step, 0)
    plsc.subcore_barrier()
    pltpu.sync_copy(acc_d.at[pl.ds(row0, _STRIPE)],
                    out_d.at[c, pl.ds(row0, _STRIPE)])


_PASS_CACHE = {}


def _mesh_kw():
    return dict(mesh=plsc.VectorSubcoreMesh(core_axis_name="c",
                                            subcore_axis_name="s",
                                            num_cores=_NC, num_subcores=_NS))


def _gather_pass(*args):
    fn = _PASS_CACHE.get("g")
    if fn is None:
        fn = pl.kernel(
            _gather_body,
            out_type=jax.ShapeDtypeStruct((_NC, _NPAD, _H), jnp.float32),
            scratch_types=[
                pltpu.VMEM_SHARED((_NPAD, _H), jnp.float32),
                pltpu.VMEM((_BPH, _BATCH), jnp.int32),
                pltpu.VMEM((_BPH, _BATCH), jnp.int32),
                pltpu.VMEM((_BATCH, _H), jnp.float32),
                pltpu.VMEM((_BATCH, _H), jnp.float32),
                pltpu.SemaphoreType.DMA,
                pltpu.SemaphoreType.DMA,
            ],
            **_mesh_kw())
        _PASS_CACHE["g"] = fn
    return fn(*args)


def _deg_pass(*args):
    fn = _PASS_CACHE.get("d")
    if fn is None:
        fn = pl.kernel(
            _deg_body,
            out_type=jax.ShapeDtypeStruct((_NC, _NPAD, _H), jnp.float32),
            scratch_types=[
                pltpu.VMEM_SHARED((_NPAD, _H), jnp.float32),
                pltpu.VMEM((_BPW, _BATCH), jnp.int32),
                pltpu.VMEM((_BATCH, _H), jnp.float32),
            ],
            **_mesh_kw())
        _PASS_CACHE["d"] = fn
    return fn(*args)


def _prep_body(x, out):
    v = x[...]
    n = jnp.sqrt(jnp.sum(v * v, axis=1, keepdims=True))
    out[...] = v / jnp.maximum(n, 1e-12)


def _prep(x):
    return pl.pallas_call(
        _prep_body,
        grid=(_GRID,),
        in_specs=[pl.BlockSpec((_RB, _H), lambda i: (i, 0))],
        out_specs=pl.BlockSpec((_RB, _H), lambda i: (i, 0)),
        out_shape=jax.ShapeDtypeStruct((_N, _H), jnp.float32),
    )(x)


def _layer_core(gp, rp, dp, cur, wn, wl, we):
    s_sum = gp[0] + gp[1] + rp[0] + rp[1]
    deg = dp[0, :, 0:1] + dp[1, :, 0:1]
    norm = 1.0 / jnp.maximum(deg, 1.0)
    agg = jnp.dot(s_sum, wn[...], preferred_element_type=jnp.float32) * norm
    cv = cur[...]
    lw = jnp.dot(cv, wl[...], preferred_element_type=jnp.float32)
    le = jnp.dot(cv, we[...], preferred_element_type=jnp.float32)
    x = agg + jnp.where(deg > 0, lw, le)
    return jnp.where(x >= 0, x, x * _SLOPE)


def _layer_body(gp, rp, dp, cur, wn, wl, we, out):
    out[...] = _layer_core(gp, rp, dp, cur, wn, wl, we)


def _final_body(gp, rp, dp, cur, wn, wl, we, h, wtg, btg, out):
    cur2 = _layer_core(gp, rp, dp, cur, wn, wl, we)
    n2 = jnp.sqrt(jnp.sum(cur2 * cur2, axis=1, keepdims=True))
    curn = cur2 / jnp.maximum(n2, 1e-12)
    hv = h[...]
    tw = jax.nn.sigmoid(
        jnp.dot(hv, wtg[...], preferred_element_type=jnp.float32) + btg[...])
    hn = tw * curn + (1.0 - tw) * hv
    nh = jnp.sqrt(jnp.sum(hn * hn, axis=1, keepdims=True))
    out[...] = hn / jnp.maximum(nh, 1e-12)


_SPEC_G = pl.BlockSpec((_NC, _RB, _H), lambda i: (0, i, 0))
_SPEC_D = pl.BlockSpec((_NC, _RB, _H), lambda i: (0, i, 0))
_SPEC_X = pl.BlockSpec((_RB, _H), lambda i: (i, 0))
_SPEC_W = pl.BlockSpec((_H, _H), lambda i: (0, 0))
_SPEC_B = pl.BlockSpec((1, _H), lambda i: (0, 0))


def _layer(gp, rp, dp, cur, wn, wl, we):
    return pl.pallas_call(
        _layer_body,
        grid=(_GRID,),
        in_specs=[_SPEC_G, _SPEC_G, _SPEC_D, _SPEC_X, _SPEC_W, _SPEC_W,
                  _SPEC_W],
        out_specs=_SPEC_X,
        out_shape=jax.ShapeDtypeStruct((_N, _H), jnp.float32),
    )(gp, rp, dp, cur, wn, wl, we)


def _final(gp, rp, dp, cur, wn, wl, we, h, wtg, btg):
    return pl.pallas_call(
        _final_body,
        grid=(_GRID,),
        in_specs=[_SPEC_G, _SPEC_G, _SPEC_D, _SPEC_X, _SPEC_W, _SPEC_W,
                  _SPEC_W, _SPEC_X, _SPEC_W, _SPEC_B],
        out_specs=_SPEC_X,
        out_shape=jax.ShapeDtypeStruct((_N, _H), jnp.float32),
    )(gp, rp, dp, cur, wn, wl, we, h, wtg, btg)


def kernel(edge_src, edge_dst, edge_type, dynamic_emb, emb_rel,
           weight_neighbor, loop_weight, evolve_loop_weight,
           time_gate_weight, time_gate_bias):
    f32 = jnp.float32
    src = edge_src.astype(jnp.int32)
    dst = edge_dst.astype(jnp.int32)
    ety = edge_type.astype(jnp.int32)
    padlen = _EPAD - _E
    pad0 = jnp.zeros((_T, padlen), jnp.int32)
    trash = _N + (jnp.arange(padlen, dtype=jnp.int32) % 16)
    src_p = jnp.concatenate([src, pad0], axis=1).reshape(_T, _NB, _BATCH)
    ety_p = jnp.concatenate([ety, pad0], axis=1).reshape(_T, _NB, _BATCH)
    dst_p = jnp.concatenate(
        [dst, jnp.broadcast_to(trash, (_T, padlen))], axis=1
    ).reshape(_T, _NB, _BATCH)

    zeros_g = jnp.zeros((_NPAD, _H), f32)
    ones_hbm = jnp.ones((_BATCH, _H), f32)
    emb_rel_f = emb_rel.astype(f32)
    btg = time_gate_bias.astype(f32).reshape(1, _H)

    h = _prep(dynamic_emb.astype(f32))
    outs = []
    for t in range(_T):
        rp = _gather_pass(emb_rel_f, ety_p[t], dst_p[t], zeros_g)
        dp = _deg_pass(dst_p[t], zeros_g, ones_hbm)
        gp = _gather_pass(h, src_p[t], dst_p[t], zeros_g)
        cur1 = _layer(gp, rp, dp, h, weight_neighbor[0], loop_weight[0],
                      evolve_loop_weight[0])
        gp2 = _gather_pass(cur1, src_p[t], dst_p[t], zeros_g)
        h = _final(gp2, rp, dp, cur1, weight_neighbor[1], loop_weight[1],
                   evolve_loop_weight[1], h, time_gate_weight, btg)
        outs.append(h)
    return (jnp.stack(outs, axis=0), emb_rel)
